# Initial kernel scaffold; baseline (speedup 1.0000x reference)
import jax
import jax.numpy as jnp
from jax.experimental import pallas as pl


def kernel(x, table):
    raise NotImplementedError("write your pallas kernel here")


# speedup vs baseline: 4.5389x; 4.5389x over previous
import functools

import jax
import jax.numpy as jnp
from jax import lax
from jax.experimental import pallas as pl
from jax.experimental.pallas import tpu as pltpu
from jax.experimental.pallas import tpu_sc as plsc

EMBED_DIM = 64
BATCH = 4096
HIST = 50

NC = 2
NS = 16
NW = NC * NS
TOTAL = BATCH * HIST
CHUNK = 128
ROWS_PER_W = TOTAL // NW
CHUNKS_PER_W = ROWS_PER_W // CHUNK


def _emb_body(idx_hbm, table_hbm, out_hbm, idx_v, rows_v, sem0, sem1):
    wid = lax.axis_index("s") * NC + lax.axis_index("c")
    base = wid * ROWS_PER_W
    pltpu.sync_copy(idx_hbm.at[wid], idx_v)

    sems = (sem0, sem1)

    def gather(j, b):
        return pltpu.make_async_copy(
            table_hbm.at[idx_v.at[j]], rows_v.at[b], sems[b]
        )

    gather(0, 0).start()

    def body(jj, _):
        for b in range(2):
            j = jj * 2 + b

            @pl.when(j + 1 < CHUNKS_PER_W)
            def _():
                gather(j + 1, 1 - b).start()

            gather(j, b).wait()
            pltpu.sync_copy(
                rows_v.at[b], out_hbm.at[pl.ds(base + j * CHUNK, CHUNK)]
            )
        return 0

    lax.fori_loop(0, CHUNKS_PER_W // 2, body, 0)


@functools.partial(jax.jit)
def kernel(x, table):
    idx = x.astype(jnp.int32).reshape(NW, CHUNKS_PER_W, CHUNK)
    mesh = plsc.VectorSubcoreMesh(core_axis_name="c", subcore_axis_name="s")
    out = pl.kernel(
        _emb_body,
        out_type=jax.ShapeDtypeStruct((TOTAL, EMBED_DIM), jnp.float32),
        mesh=mesh,
        scratch_types=[
            pltpu.VMEM((CHUNKS_PER_W, CHUNK), jnp.int32),
            pltpu.VMEM((2, CHUNK, EMBED_DIM), jnp.float32),
            pltpu.SemaphoreType.DMA,
            pltpu.SemaphoreType.DMA,
        ],
        compiler_params=pltpu.CompilerParams(use_tc_tiling_on_sc=False),
    )(idx, table)
    return out.reshape(BATCH, HIST, EMBED_DIM)


# speedup vs baseline: 4.6781x; 1.0307x over previous
import functools

import jax
import jax.numpy as jnp
from jax import lax
from jax.experimental import pallas as pl
from jax.experimental.pallas import tpu as pltpu
from jax.experimental.pallas import tpu_sc as plsc

EMBED_DIM = 64
BATCH = 4096
HIST = 50

NC = 2
NS = 16
NW = NC * NS
TOTAL = BATCH * HIST
CHUNK = 128
ROWS_PER_W = TOTAL // NW
CHUNKS_PER_W = ROWS_PER_W // CHUNK


NBUF = 4
MAIN_CHUNKS = (CHUNKS_PER_W // NBUF) * NBUF


def _emb_body(idx_hbm, table_hbm, out_hbm, idx_v, rows_v,
              g0, g1, g2, g3, w0, w1, w2, w3):
    wid = lax.axis_index("s") * NC + lax.axis_index("c")
    base = wid * ROWS_PER_W
    pltpu.sync_copy(idx_hbm.at[wid], idx_v)

    gs = (g0, g1, g2, g3)
    ws = (w0, w1, w2, w3)

    def gather(j, b):
        return pltpu.make_async_copy(
            table_hbm.at[idx_v.at[j]], rows_v.at[b], gs[b]
        )

    def write(j, b):
        return pltpu.make_async_copy(
            rows_v.at[b], out_hbm.at[pl.ds(base + j * CHUNK, CHUNK)], ws[b]
        )

    for b in range(NBUF):
        gather(b, b).start()

    def body(jj, _):
        for b in range(NBUF):
            j = jj * NBUF + b
            gather(j, b).wait()
            write(j, b).start()
            prev_b = (b - 1) % NBUF

            @pl.when(jnp.logical_and(j >= 1, j + NBUF - 1 < CHUNKS_PER_W))
            def _():
                write(j - 1, prev_b).wait()
                gather(j + NBUF - 1, prev_b).start()
        return 0

    lax.fori_loop(0, MAIN_CHUNKS // NBUF, body, 0)

    for j in range(MAIN_CHUNKS, CHUNKS_PER_W):
        b = j % NBUF
        gather(j, b).wait()
        write(j, b).start()
    for j in range(CHUNKS_PER_W - NBUF, CHUNKS_PER_W):
        write(j, j % NBUF).wait()


@functools.partial(jax.jit)
def kernel(x, table):
    idx = x.astype(jnp.int32).reshape(NW, CHUNKS_PER_W, CHUNK)
    mesh = plsc.VectorSubcoreMesh(core_axis_name="c", subcore_axis_name="s")
    out = pl.kernel(
        _emb_body,
        out_type=jax.ShapeDtypeStruct((TOTAL, EMBED_DIM), jnp.float32),
        mesh=mesh,
        scratch_types=[
            pltpu.VMEM((CHUNKS_PER_W, CHUNK), jnp.int32),
            pltpu.VMEM((NBUF, CHUNK, EMBED_DIM), jnp.float32),
        ] + [pltpu.SemaphoreType.DMA] * (2 * NBUF),
        compiler_params=pltpu.CompilerParams(use_tc_tiling_on_sc=False),
    )(idx, table)
    return out.reshape(BATCH, HIST, EMBED_DIM)


# speedup vs baseline: 4.6789x; 1.0002x over previous
import functools

import jax
import jax.numpy as jnp
from jax import lax
from jax.experimental import pallas as pl
from jax.experimental.pallas import tpu as pltpu
from jax.experimental.pallas import tpu_sc as plsc

EMBED_DIM = 64
BATCH = 4096
HIST = 50

NC = 2
NS = 16
NW = NC * NS
TOTAL = BATCH * HIST
CHUNK = 256
ROWS_PER_W = TOTAL // NW
CHUNKS_PER_W = ROWS_PER_W // CHUNK


NBUF = 4
MAIN_CHUNKS = (CHUNKS_PER_W // NBUF) * NBUF


def _emb_body(idx_hbm, table_hbm, out_hbm, idx_v, rows_v,
              g0, g1, g2, g3, w0, w1, w2, w3):
    wid = lax.axis_index("s") * NC + lax.axis_index("c")
    base = wid * ROWS_PER_W
    pltpu.sync_copy(idx_hbm.at[wid], idx_v)

    gs = (g0, g1, g2, g3)
    ws = (w0, w1, w2, w3)

    def gather(j, b):
        return pltpu.make_async_copy(
            table_hbm.at[idx_v.at[j]], rows_v.at[b], gs[b]
        )

    def write(j, b):
        return pltpu.make_async_copy(
            rows_v.at[b], out_hbm.at[pl.ds(base + j * CHUNK, CHUNK)], ws[b]
        )

    for b in range(NBUF):
        gather(b, b).start()

    def body(jj, _):
        for b in range(NBUF):
            j = jj * NBUF + b
            gather(j, b).wait()
            write(j, b).start()
            prev_b = (b - 1) % NBUF

            @pl.when(jnp.logical_and(j >= 1, j + NBUF - 1 < CHUNKS_PER_W))
            def _():
                write(j - 1, prev_b).wait()
                gather(j + NBUF - 1, prev_b).start()
        return 0

    lax.fori_loop(0, MAIN_CHUNKS // NBUF, body, 0)

    for j in range(MAIN_CHUNKS, CHUNKS_PER_W):
        b = j % NBUF
        gather(j, b).wait()
        write(j, b).start()
    for j in range(CHUNKS_PER_W - NBUF, CHUNKS_PER_W):
        write(j, j % NBUF).wait()


@functools.partial(jax.jit)
def kernel(x, table):
    idx = x.astype(jnp.int32).reshape(NW, CHUNKS_PER_W, CHUNK)
    mesh = plsc.VectorSubcoreMesh(core_axis_name="c", subcore_axis_name="s")
    out = pl.kernel(
        _emb_body,
        out_type=jax.ShapeDtypeStruct((TOTAL, EMBED_DIM), jnp.float32),
        mesh=mesh,
        scratch_types=[
            pltpu.VMEM((CHUNKS_PER_W, CHUNK), jnp.int32),
            pltpu.VMEM((NBUF, CHUNK, EMBED_DIM), jnp.float32),
        ] + [pltpu.SemaphoreType.DMA] * (2 * NBUF),
        compiler_params=pltpu.CompilerParams(use_tc_tiling_on_sc=False),
    )(idx, table)
    return out.reshape(BATCH, HIST, EMBED_DIM)
